# Initial kernel scaffold; baseline (speedup 1.0000x reference)
import jax
import jax.numpy as jnp
from jax.experimental import pallas as pl


def kernel(x, table):
    raise NotImplementedError("write your pallas kernel here")


# speedup vs baseline: 4.0865x; 4.0865x over previous
import functools

import jax
import jax.numpy as jnp
from jax import lax
from jax.experimental import pallas as pl
from jax.experimental.pallas import tpu as pltpu
from jax.experimental.pallas import tpu_sc as plsc

EMBED_DIM = 64
NUM_CORES = 2
NUM_SUBCORES = 16
NW = NUM_CORES * NUM_SUBCORES
CHUNK = 128

_mesh = plsc.VectorSubcoreMesh(core_axis_name="c", subcore_axis_name="s")


def _make_gather(batch: int, dim: int):
  bpw = batch // NW
  nchunk = bpw // CHUNK

  @functools.partial(
      pl.kernel,
      mesh=_mesh,
      compiler_params=pltpu.CompilerParams(use_tc_tiling_on_sc=False),
      out_type=jax.ShapeDtypeStruct((batch, dim), jnp.float32),
      scratch_types=[
          pltpu.VMEM((nchunk, CHUNK), jnp.int32),
          pltpu.VMEM((CHUNK, dim), jnp.float32),
          pltpu.SemaphoreType.DMA,
      ],
  )
  def gather_kernel(idx_hbm, table_hbm, out_hbm, idx_v, buf, sem):
    wid = lax.axis_index("s") * NUM_CORES + lax.axis_index("c")
    base = wid * bpw
    pltpu.sync_copy(idx_hbm.at[wid], idx_v)

    def body(j, carry):
      pltpu.async_copy(table_hbm.at[idx_v.at[j]], buf, sem).wait()
      pltpu.sync_copy(buf, out_hbm.at[pl.ds(base + j * CHUNK, CHUNK)])
      return carry

    lax.fori_loop(0, nchunk, body, 0)

  return gather_kernel


def kernel(x, table):
  batch, hist = x.shape
  total = batch * hist
  idx = x.reshape(NW, total // (NW * CHUNK), CHUNK).astype(jnp.int32)
  out = _make_gather(total, EMBED_DIM)(idx, table)
  return out.reshape(batch, hist, EMBED_DIM)


# speedup vs baseline: 4.6848x; 1.1464x over previous
import functools

import jax
import jax.numpy as jnp
from jax import lax
from jax.experimental import pallas as pl
from jax.experimental.pallas import tpu as pltpu
from jax.experimental.pallas import tpu_sc as plsc

EMBED_DIM = 64
NUM_CORES = 2
NUM_SUBCORES = 16
NW = NUM_CORES * NUM_SUBCORES
CHUNK = 128
NBUF = 10
LAG = 5

_mesh = plsc.VectorSubcoreMesh(core_axis_name="c", subcore_axis_name="s")


def _make_gather(batch: int, dim: int):
  bpw = batch // NW
  nchunk = bpw // CHUNK
  ngroup = nchunk // NBUF
  assert nchunk % NBUF == 0

  @functools.partial(
      pl.kernel,
      mesh=_mesh,
      compiler_params=pltpu.CompilerParams(use_tc_tiling_on_sc=False),
      out_type=jax.ShapeDtypeStruct((batch, dim), jnp.float32),
      scratch_types=[
          pltpu.VMEM((nchunk, CHUNK), jnp.int32),
          pltpu.VMEM((NBUF, CHUNK, dim), jnp.float32),
      ]
      + [pltpu.SemaphoreType.DMA] * (2 * NBUF),
  )
  def gather_kernel(idx_hbm, table_hbm, out_hbm, idx_v, buf, *sems):
    gsem = sems[:NBUF]
    osem = sems[NBUF:]
    wid = lax.axis_index("s") * NUM_CORES + lax.axis_index("c")
    base = wid * bpw
    pltpu.sync_copy(idx_hbm.at[wid], idx_v)

    def fire(j, b):
      pltpu.async_copy(table_hbm.at[idx_v.at[j]], buf.at[b], gsem[b])

    def drain(j, b):
      pltpu.make_async_copy(
          table_hbm.at[idx_v.at[j]], buf.at[b], gsem[b]
      ).wait()
      pltpu.async_copy(
          buf.at[b], out_hbm.at[pl.ds(base + j * CHUNK, CHUNK)], osem[b]
      )

    def wait_out(j, b):
      pltpu.make_async_copy(
          buf.at[b], out_hbm.at[pl.ds(base + j * CHUNK, CHUNK)], osem[b]
      ).wait()

    def body(g, carry):
      for b in range(NBUF):
        i = g * NBUF + b
        pl.when(g >= 1)(lambda: wait_out(i - NBUF, b))
        fire(i, b)
        b2 = (b + LAG) % NBUF
        if b < LAG:
          pl.when(g >= 1)(lambda: drain(i - LAG, b2))
        else:
          drain(i - LAG, b2)
      return carry

    lax.fori_loop(0, ngroup, body, 0, unroll=False)

    last = ngroup - 1
    for b in range(LAG):
      j = last * NBUF + NBUF - LAG + b
      drain(j, b + NBUF - LAG)
    for b in range(NBUF):
      j = last * NBUF + b
      wait_out(j, b)

  return gather_kernel


def kernel(x, table):
  batch, hist = x.shape
  total = batch * hist
  idx = x.reshape(NW, total // (NW * CHUNK), CHUNK).astype(jnp.int32)
  out = _make_gather(total, EMBED_DIM)(idx, table)
  return out.reshape(batch, hist, EMBED_DIM)
